# baseline (speedup 1.0000x reference)
import functools

import jax
import jax.numpy as jnp
from jax import lax
from jax.experimental import pallas as pl
from jax.experimental.pallas import tpu as pltpu
from jax.experimental.pallas import tpu_sc as plsc

B = 16384
D = 64
REG_COEF = 1e-12
NC = 2
NS = 16
NW = NC * NS
L = 16
BPW = B // NW
NCHUNK = 4
CROWS = BPW // NCHUNK


def _sc_body(uids, iids, utab, itab, preds, psum_u, psum_i,
             ids_u_v, ids_i_v, rows_u, rows_i, out_v, acc_v, sem0, sem1):
    wid = lax.axis_index("s") * NC + lax.axis_index("c")
    base = wid * BPW

    pltpu.sync_copy(uids.at[pl.ds(base, BPW)], ids_u_v)
    pltpu.sync_copy(iids.at[pl.ds(base, BPW)], ids_i_v)

    def fire(c, sems):
        buf = c % 2
        sem = sems[buf]

        def fire_group(g, carry):
            gbase = c * CROWS + g * L
            uvec = ids_u_v[pl.ds(gbase, L)]
            ivec = ids_i_v[pl.ds(gbase, L)]
            for k in range(L):
                pltpu.async_copy(utab.at[uvec[k]], rows_u.at[buf, g * L + k], sem)
                pltpu.async_copy(itab.at[ivec[k]], rows_i.at[buf, g * L + k], sem)
            return carry

        lax.fori_loop(0, CROWS // L, fire_group, 0)

    def drain(c, sems):
        buf = c % 2
        sem = sems[buf]
        pltpu.make_async_copy(utab.at[pl.ds(0, CROWS)], rows_u.at[buf], sem).wait()
        pltpu.make_async_copy(itab.at[pl.ds(0, CROWS)], rows_i.at[buf], sem).wait()

    lane = lax.iota(jnp.int32, L)

    def hsum(x):
        for sft in (8, 4, 2, 1):
            x = x + x.at[lane ^ sft].get(mode="promise_in_bounds")
        return x

    def compute(c, carry):
        buf = c % 2

        def group_body(g, carry):
            su, si = carry
            res = jnp.zeros((L,), jnp.float32)
            for k in range(L):
                r = g * L + k
                acc = None
                for j in range(D // L):
                    u = rows_u[buf, r, pl.ds(j * L, L)]
                    v = rows_i[buf, r, pl.ds(j * L, L)]
                    acc = u * v if acc is None else acc + u * v
                    su = su + u * u
                    si = si + v * v
                res = jnp.where(lane == k, hsum(acc), res)
            out_v[pl.ds(c * CROWS + g * L, L)] = res
            return su, si

        return lax.fori_loop(0, CROWS // L, group_body, carry)

    sems = (sem0, sem1)
    zero = jnp.zeros((L,), jnp.float32)
    carry = (zero, zero)
    fire(0, sems)
    for c in range(NCHUNK):
        if c + 1 < NCHUNK:
            fire(c + 1, sems)
        drain(c, sems)
        carry = compute(c, carry)
    su, si = carry

    acc_v[0, :] = su
    acc_v[1, :] = si
    pltpu.sync_copy(out_v, preds.at[pl.ds(base, BPW)])
    pltpu.sync_copy(acc_v.at[0], psum_u.at[wid])
    pltpu.sync_copy(acc_v.at[1], psum_i.at[wid])


def _epilogue_body(preds_ref, pu_ref, pi_ref, out_ref):
    su = jnp.sum(pu_ref[...])
    si = jnp.sum(pi_ref[...])
    reg = REG_COEF * (jnp.sqrt(su) + jnp.sqrt(si))
    out_ref[...] = preds_ref[...] + reg


@jax.jit
def kernel(user_ids, item_ids, user_table, item_table):
    mesh = plsc.VectorSubcoreMesh(core_axis_name="c", subcore_axis_name="s")
    sc_fn = pl.kernel(
        _sc_body,
        out_type=[
            jax.ShapeDtypeStruct((B,), jnp.float32),
            jax.ShapeDtypeStruct((NW, L), jnp.float32),
            jax.ShapeDtypeStruct((NW, L), jnp.float32),
        ],
        mesh=mesh,
        scratch_types=[
            pltpu.VMEM((BPW,), jnp.int32),
            pltpu.VMEM((BPW,), jnp.int32),
            pltpu.VMEM((2, CROWS, D), jnp.float32),
            pltpu.VMEM((2, CROWS, D), jnp.float32),
            pltpu.VMEM((BPW,), jnp.float32),
            pltpu.VMEM((2, L), jnp.float32),
            pltpu.SemaphoreType.DMA,
            pltpu.SemaphoreType.DMA,
        ],
    )
    preds_raw, pu, pi = sc_fn(user_ids, item_ids, user_table, item_table)

    preds2d = preds_raw.reshape(B // 128, 128)
    out2d = pl.pallas_call(
        _epilogue_body,
        out_shape=jax.ShapeDtypeStruct((B // 128, 128), jnp.float32),
    )(preds2d, pu, pi)
    return out2d.reshape(B)


# speedup vs baseline: 2.0195x; 2.0195x over previous
import jax
import jax.numpy as jnp
from jax import lax
from jax.experimental import pallas as pl
from jax.experimental.pallas import tpu as pltpu
from jax.experimental.pallas import tpu_sc as plsc

B = 16384
D = 64
REG_COEF = 1e-12
NC = 2
NS = 16
NW = NC * NS
L = 16
BPW = B // NW
BLK = 128
NCHUNK = 4
CROWS = BPW // NCHUNK


def _gather1(ref, p):
    return plsc.load_gather(ref, [jnp.full((L,), p, jnp.int32)])[0]


def _phase1_body(us, up, is_, ip_, utab, itab, gu, gi,
                 sidv, permv, dblk, dstart, blockbuf, staging,
                 bsem0, bsem1, osem):
    wid = lax.axis_index("s") * NC + lax.axis_index("c")
    base = wid * BPW
    lane = lax.iota(jnp.int32, L)

    def do_table(sort_hbm, perm_hbm, tab, gout):
        pltpu.sync_copy(sort_hbm.at[pl.ds(base, BPW)], sidv)
        pltpu.sync_copy(perm_hbm.at[pl.ds(base, BPW)], permv)

        def iscan(x):
            for sft in (1, 2, 4, 8):
                sh = x.at[jnp.maximum(lane - sft, 0)].get(
                    mode="promise_in_bounds")
                x = x + jnp.where(lane >= sft, sh, 0)
            return x

        def scan_w(w, carry):
            cnt, prev = carry
            bv = sidv[pl.ds(w * L, L)] >> 7
            shifted = bv.at[jnp.maximum(lane - 1, 0)].get(
                mode="promise_in_bounds")
            pv = jnp.where(lane == 0, prev, shifted)
            m = bv != pv
            mi = m.astype(jnp.int32)
            incl = iscan(mi)
            dest = jnp.where(m, cnt + incl - mi, BPW + 2 * L - 1)
            plsc.store_scatter(dblk, [dest], bv)
            plsc.store_scatter(dstart, [dest], w * L + lane)
            return cnt + incl[L - 1], bv[L - 1]

        nd, _ = lax.fori_loop(0, BPW // L, scan_w,
                              (jnp.int32(0), jnp.int32(-1)))
        plsc.store_scatter(dstart, [jnp.full((L,), nd, jnp.int32)],
                           jnp.full((L,), BPW, jnp.int32))

        def fire_blk(d, buf, sem):
            col0 = pl.multiple_of(_gather1(dblk, d) * BLK, BLK)
            pltpu.async_copy(tab.at[:, pl.ds(col0, BLK)],
                             blockbuf.at[buf], sem)

        def drain_blk(buf, sem):
            pltpu.make_async_copy(tab.at[:, pl.ds(0, BLK)],
                                  blockbuf.at[buf], sem).wait()

        def process_run(d, buf):
            s = _gather1(dstart, d)
            e = _gather1(dstart, d + 1)

            def member(p, carry):
                sid = _gather1(sidv, p)
                b = _gather1(permv, p)
                ci = jnp.full((L,), sid & (BLK - 1), jnp.int32)
                for j in range(D // L):
                    colv = plsc.load_gather(blockbuf.at[buf],
                                            [j * L + lane, ci])
                    staging[pl.ds(p * D + j * L, L)] = colv
                pltpu.async_copy(staging.at[pl.ds(p * D, D)],
                                 gout.at[pl.ds(b * D, D)], osem)
                return carry

            lax.fori_loop(s, e, member, 0)

        fire_blk(0, 0, bsem0)

        def pair_body(h, carry):
            d0 = 2 * h

            @pl.when(d0 + 1 < nd)
            def _():
                fire_blk(d0 + 1, 1, bsem1)

            drain_blk(0, bsem0)
            process_run(d0, 0)

            @pl.when(d0 + 2 < nd)
            def _():
                fire_blk(d0 + 2, 0, bsem0)

            @pl.when(d0 + 1 < nd)
            def _():
                drain_blk(1, bsem1)
                process_run(d0 + 1, 1)

            return carry

        lax.fori_loop(0, (nd + 1) // 2, pair_body, 0)
        pltpu.make_async_copy(gout.at[pl.ds(0, BPW * D)], staging, osem).wait()

    do_table(us, up, utab, gu)
    do_table(is_, ip_, itab, gi)


def _phase2_body(gu, gi, preds, psum_u, psum_i,
                 bufu, bufi, out_v, acc_v, sem0, sem1):
    wid = lax.axis_index("s") * NC + lax.axis_index("c")
    base = wid * BPW
    lane = lax.iota(jnp.int32, L)
    sems = (sem0, sem1)

    def fire(c):
        buf = c % 2
        off = (base + c * CROWS) * D
        pltpu.async_copy(gu.at[pl.ds(off, CROWS * D)], bufu.at[buf], sems[buf])
        pltpu.async_copy(gi.at[pl.ds(off, CROWS * D)], bufi.at[buf], sems[buf])

    def drain(c):
        buf = c % 2
        pltpu.make_async_copy(gu.at[pl.ds(0, CROWS * D)],
                              bufu.at[buf], sems[buf]).wait()
        pltpu.make_async_copy(gi.at[pl.ds(0, CROWS * D)],
                              bufi.at[buf], sems[buf]).wait()

    def hsum(x):
        for sft in (8, 4, 2, 1):
            x = x + x.at[lane ^ sft].get(mode="promise_in_bounds")
        return x

    def compute(c, carry):
        buf = c % 2

        def group_body(g, carry):
            su, si = carry
            res = jnp.zeros((L,), jnp.float32)
            for k in range(L):
                r = g * L + k
                acc = None
                for j in range(D // L):
                    u = bufu[buf, pl.ds(r * D + j * L, L)]
                    v = bufi[buf, pl.ds(r * D + j * L, L)]
                    acc = u * v if acc is None else acc + u * v
                    su = su + u * u
                    si = si + v * v
                res = jnp.where(lane == k, hsum(acc), res)
            out_v[pl.ds(c * CROWS + g * L, L)] = res
            return su, si

        return lax.fori_loop(0, CROWS // L, group_body, carry)

    zero = jnp.zeros((L,), jnp.float32)
    carry = (zero, zero)
    fire(0)
    for c in range(NCHUNK):
        if c + 1 < NCHUNK:
            fire(c + 1)
        drain(c)
        carry = compute(c, carry)
    su, si = carry

    acc_v[0, :] = su
    acc_v[1, :] = si
    pltpu.sync_copy(out_v, preds.at[pl.ds(base, BPW)])
    pltpu.sync_copy(acc_v.at[0], psum_u.at[wid])
    pltpu.sync_copy(acc_v.at[1], psum_i.at[wid])


def _epilogue_body(preds_ref, pu_ref, pi_ref, out_ref):
    su = jnp.sum(pu_ref[...])
    si = jnp.sum(pi_ref[...])
    reg = REG_COEF * (jnp.sqrt(su) + jnp.sqrt(si))
    out_ref[...] = preds_ref[...] + reg


@jax.jit
def kernel(user_ids, item_ids, user_table, item_table):
    mesh = plsc.VectorSubcoreMesh(core_axis_name="c", subcore_axis_name="s")

    phase1 = pl.kernel(
        _phase1_body,
        out_type=[
            jax.ShapeDtypeStruct((B * D,), jnp.float32),
            jax.ShapeDtypeStruct((B * D,), jnp.float32),
        ],
        mesh=mesh,
        scratch_types=[
            pltpu.VMEM((BPW,), jnp.int32),
            pltpu.VMEM((BPW,), jnp.int32),
            pltpu.VMEM((BPW + 2 * L,), jnp.int32),
            pltpu.VMEM((BPW + 2 * L,), jnp.int32),
            pltpu.VMEM((2, D, BLK), jnp.float32),
            pltpu.VMEM((BPW * D,), jnp.float32),
            pltpu.SemaphoreType.DMA,
            pltpu.SemaphoreType.DMA,
            pltpu.SemaphoreType.DMA,
        ],
        compiler_params=pltpu.CompilerParams(needs_layout_passes=False),
    )

    phase2 = pl.kernel(
        _phase2_body,
        out_type=[
            jax.ShapeDtypeStruct((B,), jnp.float32),
            jax.ShapeDtypeStruct((NW, L), jnp.float32),
            jax.ShapeDtypeStruct((NW, L), jnp.float32),
        ],
        mesh=mesh,
        scratch_types=[
            pltpu.VMEM((2, CROWS * D), jnp.float32),
            pltpu.VMEM((2, CROWS * D), jnp.float32),
            pltpu.VMEM((BPW,), jnp.float32),
            pltpu.VMEM((2, L), jnp.float32),
            pltpu.SemaphoreType.DMA,
            pltpu.SemaphoreType.DMA,
        ],
        compiler_params=pltpu.CompilerParams(needs_layout_passes=False),
    )

    up = jnp.argsort(user_ids).astype(jnp.int32)
    us = jnp.take(user_ids, up)
    ipm = jnp.argsort(item_ids).astype(jnp.int32)
    is_ = jnp.take(item_ids, ipm)

    gu, gi = phase1(us, up, is_, ipm, user_table.T, item_table.T)
    preds_raw, pu, pi = phase2(gu, gi)

    preds2d = preds_raw.reshape(B // 128, 128)
    out2d = pl.pallas_call(
        _epilogue_body,
        out_shape=jax.ShapeDtypeStruct((B // 128, 128), jnp.float32),
    )(preds2d, pu, pi)
    return out2d.reshape(B)


# speedup vs baseline: 2.1067x; 1.0432x over previous
import jax
import jax.numpy as jnp
from jax import lax
from jax.experimental import pallas as pl
from jax.experimental.pallas import tpu as pltpu
from jax.experimental.pallas import tpu_sc as plsc

B = 16384
D = 64
REG_COEF = 1e-12
NC = 2
NS = 16
NW = NC * NS
L = 16
BPW = B // NW
BLK = 128
NCHUNK = 4
CROWS = BPW // NCHUNK


def _gather1(ref, p):
    return plsc.load_gather(ref, [jnp.full((L,), p, jnp.int32)])[0]


def _phase1_body(us, up, is_, ip_, utab, itab, gu, gi,
                 sidv, permv, dblk, dstart, blockbuf, staging,
                 bsem0, bsem1, osem):
    wid = lax.axis_index("s") * NC + lax.axis_index("c")
    base = wid * BPW
    lane = lax.iota(jnp.int32, L)

    def do_table(sort_hbm, perm_hbm, tab, gout):
        pltpu.sync_copy(sort_hbm.at[pl.ds(base, BPW)], sidv)
        pltpu.sync_copy(perm_hbm.at[pl.ds(base, BPW)], permv)

        def iscan(x):
            for sft in (1, 2, 4, 8):
                sh = x.at[jnp.maximum(lane - sft, 0)].get(
                    mode="promise_in_bounds")
                x = x + jnp.where(lane >= sft, sh, 0)
            return x

        def scan_w(w, carry):
            cnt, prev = carry
            bv = sidv[pl.ds(w * L, L)] >> 7
            shifted = bv.at[jnp.maximum(lane - 1, 0)].get(
                mode="promise_in_bounds")
            pv = jnp.where(lane == 0, prev, shifted)
            m = bv != pv
            mi = m.astype(jnp.int32)
            incl = iscan(mi)
            dest = jnp.where(m, cnt + incl - mi, BPW + 2 * L - 1)
            plsc.store_scatter(dblk, [dest], bv)
            plsc.store_scatter(dstart, [dest], w * L + lane)
            return cnt + incl[L - 1], bv[L - 1]

        nd, _ = lax.fori_loop(0, BPW // L, scan_w,
                              (jnp.int32(0), jnp.int32(-1)))
        plsc.store_scatter(dstart, [jnp.full((L,), nd, jnp.int32)],
                           jnp.full((L,), BPW, jnp.int32))

        def fire_blk(d, buf, sem):
            col0 = pl.multiple_of(_gather1(dblk, d) * BLK, BLK)
            pltpu.async_copy(tab.at[:, pl.ds(col0, BLK)],
                             blockbuf.at[buf], sem)

        def drain_blk(buf, sem):
            pltpu.make_async_copy(tab.at[:, pl.ds(0, BLK)],
                                  blockbuf.at[buf], sem).wait()

        def process_run(d, buf):
            s = _gather1(dstart, d)
            e = _gather1(dstart, d + 1)

            def member(p, carry):
                sid = _gather1(sidv, p)
                b = _gather1(permv, p)
                ci = jnp.full((L,), sid & (BLK - 1), jnp.int32)
                for j in range(D // L):
                    colv = plsc.load_gather(blockbuf.at[buf],
                                            [j * L + lane, ci])
                    staging[pl.ds(p * D + j * L, L)] = colv
                pltpu.async_copy(staging.at[pl.ds(p * D, D)],
                                 gout.at[pl.ds(b * D, D)], osem)
                return carry

            lax.fori_loop(s, e, member, 0)

        fire_blk(0, 0, bsem0)

        def pair_body(h, carry):
            d0 = 2 * h

            @pl.when(d0 + 1 < nd)
            def _():
                fire_blk(d0 + 1, 1, bsem1)

            drain_blk(0, bsem0)
            process_run(d0, 0)

            @pl.when(d0 + 2 < nd)
            def _():
                fire_blk(d0 + 2, 0, bsem0)

            @pl.when(d0 + 1 < nd)
            def _():
                drain_blk(1, bsem1)
                process_run(d0 + 1, 1)

            return carry

        lax.fori_loop(0, (nd + 1) // 2, pair_body, 0)
        pltpu.make_async_copy(gout.at[pl.ds(0, BPW * D)], staging, osem).wait()

    do_table(us, up, utab, gu)
    do_table(is_, ip_, itab, gi)


def _phase2_body(gu, gi, preds, psum_u, psum_i,
                 bufu, bufi, out_v, acc_v, sem0, sem1):
    wid = lax.axis_index("s") * NC + lax.axis_index("c")
    base = wid * BPW
    lane = lax.iota(jnp.int32, L)
    sems = (sem0, sem1)

    def fire(c):
        buf = c % 2
        off = (base + c * CROWS) * D
        pltpu.async_copy(gu.at[pl.ds(off, CROWS * D)], bufu.at[buf], sems[buf])
        pltpu.async_copy(gi.at[pl.ds(off, CROWS * D)], bufi.at[buf], sems[buf])

    def drain(c):
        buf = c % 2
        pltpu.make_async_copy(gu.at[pl.ds(0, CROWS * D)],
                              bufu.at[buf], sems[buf]).wait()
        pltpu.make_async_copy(gi.at[pl.ds(0, CROWS * D)],
                              bufi.at[buf], sems[buf]).wait()

    def hsum(x):
        for sft in (8, 4, 2, 1):
            x = x + x.at[lane ^ sft].get(mode="promise_in_bounds")
        return x

    def compute(c, carry):
        buf = c % 2

        def group_body(g, carry):
            su, si = carry
            res = jnp.zeros((L,), jnp.float32)
            for k in range(L):
                r = g * L + k
                acc = None
                for j in range(D // L):
                    u = bufu[buf, pl.ds(r * D + j * L, L)]
                    v = bufi[buf, pl.ds(r * D + j * L, L)]
                    acc = u * v if acc is None else acc + u * v
                    su = su + u * u
                    si = si + v * v
                res = jnp.where(lane == k, hsum(acc), res)
            out_v[pl.ds(c * CROWS + g * L, L)] = res
            return su, si

        return lax.fori_loop(0, CROWS // L, group_body, carry)

    zero = jnp.zeros((L,), jnp.float32)
    carry = (zero, zero)
    fire(0)
    for c in range(NCHUNK):
        if c + 1 < NCHUNK:
            fire(c + 1)
        drain(c)
        carry = compute(c, carry)
    su, si = carry

    acc_v[0, :] = su
    acc_v[1, :] = si
    pltpu.sync_copy(out_v, preds.at[pl.ds(base, BPW)])
    pltpu.sync_copy(acc_v.at[0], psum_u.at[wid])
    pltpu.sync_copy(acc_v.at[1], psum_i.at[wid])


def _epilogue_body(preds_ref, pu_ref, pi_ref, out_ref):
    su = jnp.sum(pu_ref[...])
    si = jnp.sum(pi_ref[...])
    reg = REG_COEF * (jnp.sqrt(su) + jnp.sqrt(si))
    out_ref[...] = preds_ref[...] + reg


@jax.jit
def kernel(user_ids, item_ids, user_table, item_table):
    mesh = plsc.VectorSubcoreMesh(core_axis_name="c", subcore_axis_name="s")

    phase1 = pl.kernel(
        _phase1_body,
        out_type=[
            jax.ShapeDtypeStruct((B * D,), jnp.float32),
            jax.ShapeDtypeStruct((B * D,), jnp.float32),
        ],
        mesh=mesh,
        scratch_types=[
            pltpu.VMEM((BPW,), jnp.int32),
            pltpu.VMEM((BPW,), jnp.int32),
            pltpu.VMEM((BPW + 2 * L,), jnp.int32),
            pltpu.VMEM((BPW + 2 * L,), jnp.int32),
            pltpu.VMEM((2, D, BLK), jnp.float32),
            pltpu.VMEM((BPW * D,), jnp.float32),
            pltpu.SemaphoreType.DMA,
            pltpu.SemaphoreType.DMA,
            pltpu.SemaphoreType.DMA,
        ],
        compiler_params=pltpu.CompilerParams(needs_layout_passes=False),
    )

    phase2 = pl.kernel(
        _phase2_body,
        out_type=[
            jax.ShapeDtypeStruct((B,), jnp.float32),
            jax.ShapeDtypeStruct((NW, L), jnp.float32),
            jax.ShapeDtypeStruct((NW, L), jnp.float32),
        ],
        mesh=mesh,
        scratch_types=[
            pltpu.VMEM((2, CROWS * D), jnp.float32),
            pltpu.VMEM((2, CROWS * D), jnp.float32),
            pltpu.VMEM((BPW,), jnp.float32),
            pltpu.VMEM((2, L), jnp.float32),
            pltpu.SemaphoreType.DMA,
            pltpu.SemaphoreType.DMA,
        ],
        compiler_params=pltpu.CompilerParams(needs_layout_passes=False),
    )

    iota = lax.iota(jnp.int32, B)
    us, up = lax.sort_key_val(user_ids, iota)
    is_, ipm = lax.sort_key_val(item_ids, iota)

    gu, gi = phase1(us, up, is_, ipm, user_table.T, item_table.T)
    preds_raw, pu, pi = phase2(gu, gi)

    preds2d = preds_raw.reshape(B // 128, 128)
    out2d = pl.pallas_call(
        _epilogue_body,
        out_shape=jax.ShapeDtypeStruct((B // 128, 128), jnp.float32),
    )(preds2d, pu, pi)
    return out2d.reshape(B)


# speedup vs baseline: 2.6725x; 1.2686x over previous
import jax
import jax.numpy as jnp
from jax import lax
from jax.experimental import pallas as pl
from jax.experimental.pallas import tpu as pltpu
from jax.experimental.pallas import tpu_sc as plsc

B = 16384
D = 64
REG_COEF = 1e-12
NC = 2
NS = 16
NW = NC * NS
L = 16
BPW = B // NW
BLK = 128
NCHUNK = 4
CROWS = BPW // NCHUNK


def _gather1(ref, p):
    return plsc.load_gather(ref, [jnp.full((L,), p, jnp.int32)])[0]


def _phase1_body(us, up, is_, ip_, utab, itab, gu, gi,
                 sidv, permv, dblk, dstart, blockbuf, staging,
                 bsem0, bsem1, bsem2, osem):
    wid = lax.axis_index("s") * NC + lax.axis_index("c")
    base = wid * BPW
    lane = lax.iota(jnp.int32, L)

    def do_table(sort_hbm, perm_hbm, tab, gout):
        pltpu.sync_copy(sort_hbm.at[pl.ds(base, BPW)], sidv)
        pltpu.sync_copy(perm_hbm.at[pl.ds(base, BPW)], permv)

        def iscan(x):
            for sft in (1, 2, 4, 8):
                sh = x.at[jnp.maximum(lane - sft, 0)].get(
                    mode="promise_in_bounds")
                x = x + jnp.where(lane >= sft, sh, 0)
            return x

        def scan_w(w, carry):
            cnt, prev = carry
            bv = sidv[pl.ds(w * L, L)] >> 7
            shifted = bv.at[jnp.maximum(lane - 1, 0)].get(
                mode="promise_in_bounds")
            pv = jnp.where(lane == 0, prev, shifted)
            m = bv != pv
            mi = m.astype(jnp.int32)
            incl = iscan(mi)
            dest = jnp.where(m, cnt + incl - mi, BPW + 2 * L - 1)
            plsc.store_scatter(dblk, [dest], bv)
            plsc.store_scatter(dstart, [dest], w * L + lane)
            return cnt + incl[L - 1], bv[L - 1]

        nd, _ = lax.fori_loop(0, BPW // L, scan_w,
                              (jnp.int32(0), jnp.int32(-1)))
        plsc.store_scatter(dstart, [jnp.full((L,), nd, jnp.int32)],
                           jnp.full((L,), BPW, jnp.int32))

        def fire_blk(d, buf, sem):
            col0 = pl.multiple_of(_gather1(dblk, d) * BLK, BLK)
            pltpu.async_copy(tab.at[:, pl.ds(col0, BLK)],
                             blockbuf.at[buf], sem)

        def drain_blk(buf, sem):
            pltpu.make_async_copy(tab.at[:, pl.ds(0, BLK)],
                                  blockbuf.at[buf], sem).wait()

        def process_run(d, buf):
            s = _gather1(dstart, d)
            e = _gather1(dstart, d + 1)

            def member(p, carry):
                sid = _gather1(sidv, p)
                b = _gather1(permv, p)
                ci = jnp.full((L,), sid & (BLK - 1), jnp.int32)
                for j in range(D // L):
                    colv = plsc.load_gather(blockbuf.at[buf],
                                            [j * L + lane, ci])
                    staging[pl.ds(p * D + j * L, L)] = colv
                pltpu.async_copy(staging.at[pl.ds(p * D, D)],
                                 gout.at[pl.ds(b * D, D)], osem)
                return carry

            lax.fori_loop(s, e, member, 0)

        fire_blk(0, 0, bsem0)

        @pl.when(1 < nd)
        def _():
            fire_blk(1, 1, bsem1)

        bsems = (bsem0, bsem1, bsem2)

        def tri_body(h, carry):
            d0 = 3 * h
            for q in range(3):
                d = d0 + q
                nbuf = (q + 2) % 3

                @pl.when(d < nd)
                def _(d=d, q=q, nbuf=nbuf):
                    @pl.when(d + 2 < nd)
                    def _():
                        fire_blk(d + 2, nbuf, bsems[nbuf])

                    drain_blk(q, bsems[q])
                    process_run(d, q)

            return carry

        lax.fori_loop(0, (nd + 2) // 3, tri_body, 0)
        pltpu.make_async_copy(gout.at[pl.ds(0, BPW * D)], staging, osem).wait()

    do_table(us, up, utab, gu)
    do_table(is_, ip_, itab, gi)


def _phase2_body(gu, gi, preds, psum_u, psum_i,
                 bufu, bufi, out_v, acc_v, sem0, sem1):
    wid = lax.axis_index("s") * NC + lax.axis_index("c")
    base = wid * BPW
    lane = lax.iota(jnp.int32, L)
    sems = (sem0, sem1)

    def fire(c):
        buf = c % 2
        off = (base + c * CROWS) * D
        pltpu.async_copy(gu.at[pl.ds(off, CROWS * D)], bufu.at[buf], sems[buf])
        pltpu.async_copy(gi.at[pl.ds(off, CROWS * D)], bufi.at[buf], sems[buf])

    def drain(c):
        buf = c % 2
        pltpu.make_async_copy(gu.at[pl.ds(0, CROWS * D)],
                              bufu.at[buf], sems[buf]).wait()
        pltpu.make_async_copy(gi.at[pl.ds(0, CROWS * D)],
                              bufi.at[buf], sems[buf]).wait()

    def hsum(x):
        for sft in (8, 4, 2, 1):
            x = x + x.at[lane ^ sft].get(mode="promise_in_bounds")
        return x

    def compute(c, carry):
        buf = c % 2

        def group_body(g, carry):
            su, si = carry
            res = jnp.zeros((L,), jnp.float32)
            for k in range(L):
                r = g * L + k
                acc = None
                for j in range(D // L):
                    u = bufu[buf, pl.ds(r * D + j * L, L)]
                    v = bufi[buf, pl.ds(r * D + j * L, L)]
                    acc = u * v if acc is None else acc + u * v
                    su = su + u * u
                    si = si + v * v
                res = jnp.where(lane == k, hsum(acc), res)
            out_v[pl.ds(c * CROWS + g * L, L)] = res
            return su, si

        return lax.fori_loop(0, CROWS // L, group_body, carry)

    zero = jnp.zeros((L,), jnp.float32)
    carry = (zero, zero)
    fire(0)
    for c in range(NCHUNK):
        if c + 1 < NCHUNK:
            fire(c + 1)
        drain(c)
        carry = compute(c, carry)
    su, si = carry

    acc_v[0, :] = su
    acc_v[1, :] = si
    pltpu.sync_copy(out_v, preds.at[pl.ds(base, BPW)])
    pltpu.sync_copy(acc_v.at[0], psum_u.at[wid])
    pltpu.sync_copy(acc_v.at[1], psum_i.at[wid])


def _epilogue_body(preds_ref, pu_ref, pi_ref, out_ref):
    su = jnp.sum(pu_ref[...])
    si = jnp.sum(pi_ref[...])
    reg = REG_COEF * (jnp.sqrt(su) + jnp.sqrt(si))
    out_ref[...] = preds_ref[...] + reg


@jax.jit
def kernel(user_ids, item_ids, user_table, item_table):
    mesh = plsc.VectorSubcoreMesh(core_axis_name="c", subcore_axis_name="s")

    phase1 = pl.kernel(
        _phase1_body,
        out_type=[
            jax.ShapeDtypeStruct((B * D,), jnp.float32),
            jax.ShapeDtypeStruct((B * D,), jnp.float32),
        ],
        mesh=mesh,
        scratch_types=[
            pltpu.VMEM((BPW,), jnp.int32),
            pltpu.VMEM((BPW,), jnp.int32),
            pltpu.VMEM((BPW + 2 * L,), jnp.int32),
            pltpu.VMEM((BPW + 2 * L,), jnp.int32),
            pltpu.VMEM((3, D, BLK), jnp.float32),
            pltpu.VMEM((BPW * D,), jnp.float32),
            pltpu.SemaphoreType.DMA,
            pltpu.SemaphoreType.DMA,
            pltpu.SemaphoreType.DMA,
            pltpu.SemaphoreType.DMA,
        ],
        compiler_params=pltpu.CompilerParams(needs_layout_passes=False),
    )

    phase2 = pl.kernel(
        _phase2_body,
        out_type=[
            jax.ShapeDtypeStruct((B,), jnp.float32),
            jax.ShapeDtypeStruct((NW, L), jnp.float32),
            jax.ShapeDtypeStruct((NW, L), jnp.float32),
        ],
        mesh=mesh,
        scratch_types=[
            pltpu.VMEM((2, CROWS * D), jnp.float32),
            pltpu.VMEM((2, CROWS * D), jnp.float32),
            pltpu.VMEM((BPW,), jnp.float32),
            pltpu.VMEM((2, L), jnp.float32),
            pltpu.SemaphoreType.DMA,
            pltpu.SemaphoreType.DMA,
        ],
        compiler_params=pltpu.CompilerParams(needs_layout_passes=False),
    )

    iota = lax.iota(jnp.int32, B)
    us, up = lax.sort_key_val(user_ids, iota)
    is_, ipm = lax.sort_key_val(item_ids, iota)

    gu, gi = phase1(us, up, is_, ipm, user_table.T, item_table.T)
    preds_raw, pu, pi = phase2(gu, gi)

    preds2d = preds_raw.reshape(B // 128, 128)
    out2d = pl.pallas_call(
        _epilogue_body,
        out_shape=jax.ShapeDtypeStruct((B // 128, 128), jnp.float32),
    )(preds2d, pu, pi)
    return out2d.reshape(B)


# speedup vs baseline: 2.6779x; 1.0020x over previous
import jax
import jax.numpy as jnp
from jax import lax
from jax.experimental import pallas as pl
from jax.experimental.pallas import tpu as pltpu
from jax.experimental.pallas import tpu_sc as plsc

B = 16384
D = 64
REG_COEF = 1e-12
NC = 2
NS = 16
NW = NC * NS
L = 16
BPW = B // NW
BLK = 128
NCHUNK = 4
CROWS = BPW // NCHUNK


def _gather1(ref, p):
    return plsc.load_gather(ref, [jnp.full((L,), p, jnp.int32)])[0]


def _phase1_body(us, up, is_, ip_, utab, itab, gu, gi,
                 sidv, permv, dblk, dstart, blockbuf, staging,
                 bsem0, bsem1, bsem2, osem):
    wid = lax.axis_index("s") * NC + lax.axis_index("c")
    base = wid * BPW
    lane = lax.iota(jnp.int32, L)

    def do_table(sort_hbm, perm_hbm, tab, gout):
        pltpu.sync_copy(sort_hbm.at[pl.ds(base, BPW)], sidv)
        pltpu.sync_copy(perm_hbm.at[pl.ds(base, BPW)], permv)

        def iscan(x):
            for sft in (1, 2, 4, 8):
                sh = x.at[jnp.maximum(lane - sft, 0)].get(
                    mode="promise_in_bounds")
                x = x + jnp.where(lane >= sft, sh, 0)
            return x

        def scan_w(w, carry):
            cnt, prev = carry
            bv = sidv[pl.ds(w * L, L)] >> 7
            shifted = bv.at[jnp.maximum(lane - 1, 0)].get(
                mode="promise_in_bounds")
            pv = jnp.where(lane == 0, prev, shifted)
            m = bv != pv
            mi = m.astype(jnp.int32)
            incl = iscan(mi)
            dest = jnp.where(m, cnt + incl - mi, BPW + 2 * L - 1)
            plsc.store_scatter(dblk, [dest], bv)
            plsc.store_scatter(dstart, [dest], w * L + lane)
            return cnt + incl[L - 1], bv[L - 1]

        nd, _ = lax.fori_loop(0, BPW // L, scan_w,
                              (jnp.int32(0), jnp.int32(-1)))
        plsc.store_scatter(dstart, [jnp.full((L,), nd, jnp.int32)],
                           jnp.full((L,), BPW, jnp.int32))

        def fire_blk(d, buf, sem):
            col0 = pl.multiple_of(_gather1(dblk, d) * BLK, BLK)
            for h in range(2):
                pltpu.async_copy(
                    tab.at[pl.ds(h * D // 2, D // 2), pl.ds(col0, BLK)],
                    blockbuf.at[buf, pl.ds(h * D // 2, D // 2)], sem)

        def drain_blk(buf, sem):
            for h in range(2):
                pltpu.make_async_copy(
                    tab.at[pl.ds(0, D // 2), pl.ds(0, BLK)],
                    blockbuf.at[buf, pl.ds(h * D // 2, D // 2)], sem).wait()

        def process_run(d, buf):
            s = _gather1(dstart, d)
            e = _gather1(dstart, d + 1)

            def member(p, carry):
                sid = _gather1(sidv, p)
                b = _gather1(permv, p)
                ci = jnp.full((L,), sid & (BLK - 1), jnp.int32)
                for j in range(D // L):
                    colv = plsc.load_gather(blockbuf.at[buf],
                                            [j * L + lane, ci])
                    staging[pl.ds(p * D + j * L, L)] = colv
                pltpu.async_copy(staging.at[pl.ds(p * D, D)],
                                 gout.at[pl.ds(b * D, D)], osem)
                return carry

            lax.fori_loop(s, e, member, 0)

        fire_blk(0, 0, bsem0)

        @pl.when(1 < nd)
        def _():
            fire_blk(1, 1, bsem1)

        bsems = (bsem0, bsem1, bsem2)

        def tri_body(h, carry):
            d0 = 3 * h
            for q in range(3):
                d = d0 + q
                nbuf = (q + 2) % 3

                @pl.when(d < nd)
                def _(d=d, q=q, nbuf=nbuf):
                    @pl.when(d + 2 < nd)
                    def _():
                        fire_blk(d + 2, nbuf, bsems[nbuf])

                    drain_blk(q, bsems[q])
                    process_run(d, q)

            return carry

        lax.fori_loop(0, (nd + 2) // 3, tri_body, 0)
        pltpu.make_async_copy(gout.at[pl.ds(0, BPW * D)], staging, osem).wait()

    do_table(us, up, utab, gu)
    do_table(is_, ip_, itab, gi)


def _phase2_body(gu, gi, preds, psum_u, psum_i,
                 bufu, bufi, out_v, acc_v, sem0, sem1):
    wid = lax.axis_index("s") * NC + lax.axis_index("c")
    base = wid * BPW
    lane = lax.iota(jnp.int32, L)
    sems = (sem0, sem1)

    def fire(c):
        buf = c % 2
        off = (base + c * CROWS) * D
        pltpu.async_copy(gu.at[pl.ds(off, CROWS * D)], bufu.at[buf], sems[buf])
        pltpu.async_copy(gi.at[pl.ds(off, CROWS * D)], bufi.at[buf], sems[buf])

    def drain(c):
        buf = c % 2
        pltpu.make_async_copy(gu.at[pl.ds(0, CROWS * D)],
                              bufu.at[buf], sems[buf]).wait()
        pltpu.make_async_copy(gi.at[pl.ds(0, CROWS * D)],
                              bufi.at[buf], sems[buf]).wait()

    def hsum(x):
        for sft in (8, 4, 2, 1):
            x = x + x.at[lane ^ sft].get(mode="promise_in_bounds")
        return x

    def compute(c, carry):
        buf = c % 2

        def group_body(g, carry):
            su, si = carry
            res = jnp.zeros((L,), jnp.float32)
            for k in range(L):
                r = g * L + k
                acc = None
                for j in range(D // L):
                    u = bufu[buf, pl.ds(r * D + j * L, L)]
                    v = bufi[buf, pl.ds(r * D + j * L, L)]
                    acc = u * v if acc is None else acc + u * v
                    su = su + u * u
                    si = si + v * v
                res = jnp.where(lane == k, hsum(acc), res)
            out_v[pl.ds(c * CROWS + g * L, L)] = res
            return su, si

        return lax.fori_loop(0, CROWS // L, group_body, carry)

    zero = jnp.zeros((L,), jnp.float32)
    carry = (zero, zero)
    fire(0)
    for c in range(NCHUNK):
        if c + 1 < NCHUNK:
            fire(c + 1)
        drain(c)
        carry = compute(c, carry)
    su, si = carry

    acc_v[0, :] = su
    acc_v[1, :] = si
    pltpu.sync_copy(out_v, preds.at[pl.ds(base, BPW)])
    pltpu.sync_copy(acc_v.at[0], psum_u.at[wid])
    pltpu.sync_copy(acc_v.at[1], psum_i.at[wid])


def _epilogue_body(preds_ref, pu_ref, pi_ref, out_ref):
    su = jnp.sum(pu_ref[...])
    si = jnp.sum(pi_ref[...])
    reg = REG_COEF * (jnp.sqrt(su) + jnp.sqrt(si))
    out_ref[...] = preds_ref[...] + reg


@jax.jit
def kernel(user_ids, item_ids, user_table, item_table):
    mesh = plsc.VectorSubcoreMesh(core_axis_name="c", subcore_axis_name="s")

    phase1 = pl.kernel(
        _phase1_body,
        out_type=[
            jax.ShapeDtypeStruct((B * D,), jnp.float32),
            jax.ShapeDtypeStruct((B * D,), jnp.float32),
        ],
        mesh=mesh,
        scratch_types=[
            pltpu.VMEM((BPW,), jnp.int32),
            pltpu.VMEM((BPW,), jnp.int32),
            pltpu.VMEM((BPW + 2 * L,), jnp.int32),
            pltpu.VMEM((BPW + 2 * L,), jnp.int32),
            pltpu.VMEM((3, D, BLK), jnp.float32),
            pltpu.VMEM((BPW * D,), jnp.float32),
            pltpu.SemaphoreType.DMA,
            pltpu.SemaphoreType.DMA,
            pltpu.SemaphoreType.DMA,
            pltpu.SemaphoreType.DMA,
        ],
        compiler_params=pltpu.CompilerParams(needs_layout_passes=False),
    )

    phase2 = pl.kernel(
        _phase2_body,
        out_type=[
            jax.ShapeDtypeStruct((B,), jnp.float32),
            jax.ShapeDtypeStruct((NW, L), jnp.float32),
            jax.ShapeDtypeStruct((NW, L), jnp.float32),
        ],
        mesh=mesh,
        scratch_types=[
            pltpu.VMEM((2, CROWS * D), jnp.float32),
            pltpu.VMEM((2, CROWS * D), jnp.float32),
            pltpu.VMEM((BPW,), jnp.float32),
            pltpu.VMEM((2, L), jnp.float32),
            pltpu.SemaphoreType.DMA,
            pltpu.SemaphoreType.DMA,
        ],
        compiler_params=pltpu.CompilerParams(needs_layout_passes=False),
    )

    iota = lax.iota(jnp.int32, B)
    us, up = lax.sort_key_val(user_ids, iota)
    is_, ipm = lax.sort_key_val(item_ids, iota)

    gu, gi = phase1(us, up, is_, ipm, user_table.T, item_table.T)
    preds_raw, pu, pi = phase2(gu, gi)

    preds2d = preds_raw.reshape(B // 128, 128)
    out2d = pl.pallas_call(
        _epilogue_body,
        out_shape=jax.ShapeDtypeStruct((B // 128, 128), jnp.float32),
    )(preds2d, pu, pi)
    return out2d.reshape(B)
